# baseline (speedup 1.0000x reference)
import functools

import jax
import jax.numpy as jnp
from jax import lax
from jax.experimental import pallas as pl
from jax.experimental.pallas import tpu as pltpu
from jax.experimental.pallas import tpu_sc as plsc

NUM_CLS = 100
HIST = NUM_CLS * NUM_CLS


def _argmax_body(yp_ref, out_ref):
    x = yp_ref[...]
    m = jnp.max(x, axis=1, keepdims=True)
    iota = lax.broadcasted_iota(jnp.int32, x.shape, 1).astype(jnp.float32)
    sel = jnp.where(x == m, iota, float(NUM_CLS))
    out_ref[...] = jnp.min(sel, axis=1, keepdims=True)


def _hist_body(nc, perw, pred_hbm, true_hbm, out_hbm, pv, tv, hist):
    wid = lax.axis_index("s") * nc + lax.axis_index("c")
    base = wid * perw

    zeros16 = jnp.zeros((16,), jnp.float32)

    def zero_body(i, carry):
        hist[pl.ds(i * 16, 16)] = zeros16
        return carry

    lax.fori_loop(0, HIST // 16, zero_body, 0)

    pltpu.sync_copy(pred_hbm.at[pl.ds(base, perw)], pv)
    pltpu.sync_copy(true_hbm.at[pl.ds(base, perw)], tv)

    ones16 = jnp.ones((16,), jnp.float32)

    def body(i, carry):
        p = pv[pl.ds(i * 16, 16)].astype(jnp.int32)
        t = tv[pl.ds(i * 16, 16)]
        k = t * NUM_CLS + p
        plsc.addupdate_scatter(hist, [k], ones16)
        return carry

    lax.fori_loop(0, perw // 16, body, 0)

    pltpu.sync_copy(hist, out_hbm.at[wid])


def _f1_body(h_ref, o_ref):
    h = h_ref[...]
    cm = jnp.sum(h, axis=0)
    ii = lax.broadcasted_iota(jnp.int32, (NUM_CLS, NUM_CLS), 0)
    jj = lax.broadcasted_iota(jnp.int32, (NUM_CLS, NUM_CLS), 1)
    diag = jnp.sum(jnp.where(ii == jj, cm, 0.0), axis=1, keepdims=True)
    rows = jnp.sum(cm, axis=1, keepdims=True)
    ones_col = jnp.ones((NUM_CLS, 1), jnp.float32)
    cols = lax.dot_general(cm, ones_col, (((0,), (0,)), ((), ())),
                           preferred_element_type=jnp.float32)
    precision = diag / (rows + 1e-12)
    recall = diag / (cols + 1e-12)
    f1 = 2.0 * precision * recall / (precision + recall + 1e-12)
    o_ref[...] = jnp.sum(f1, axis=(0, 1), keepdims=True) / NUM_CLS


def kernel(y_pred, y_true):
    n, c = y_pred.shape
    assert c == NUM_CLS

    blk = 2048
    grid = n // blk
    pred2 = pl.pallas_call(
        _argmax_body,
        grid=(grid,),
        in_specs=[pl.BlockSpec((blk, c), lambda i: (i, 0))],
        out_specs=pl.BlockSpec((blk, 1), lambda i: (i, 0)),
        out_shape=jax.ShapeDtypeStruct((n, 1), jnp.float32),
    )(y_pred)
    pred = pred2.reshape(n)

    mesh = plsc.VectorSubcoreMesh(core_axis_name="c", subcore_axis_name="s")
    nw = mesh.num_cores * mesh.num_subcores
    perw = n // nw
    hist_kernel = pl.kernel(
        functools.partial(_hist_body, mesh.num_cores, perw),
        out_type=jax.ShapeDtypeStruct((nw, HIST), jnp.float32),
        mesh=mesh,
        scratch_types=[
            pltpu.VMEM((perw,), jnp.float32),
            pltpu.VMEM((perw,), jnp.int32),
            pltpu.VMEM((HIST,), jnp.float32),
        ],
        compiler_params=pltpu.CompilerParams(needs_layout_passes=False),
    )
    hists = hist_kernel(pred, y_true)

    out = pl.pallas_call(
        _f1_body,
        out_shape=jax.ShapeDtypeStruct((1, 1), jnp.float32),
    )(hists.reshape(nw, NUM_CLS, NUM_CLS))
    return out[0, 0]


# speedup vs baseline: 3.1490x; 3.1490x over previous
import functools

import jax
import jax.numpy as jnp
from jax import lax
from jax.experimental import pallas as pl
from jax.experimental.pallas import tpu as pltpu
from jax.experimental.pallas import tpu_sc as plsc

NUM_CLS = 100
HIST = NUM_CLS * NUM_CLS


def _argmax_body(ypt_ref, yt_ref, out_ref):
    x = ypt_ref[...]
    t = yt_ref[...]
    m = jnp.max(x, axis=0, keepdims=True)
    iota = lax.broadcasted_iota(jnp.int32, x.shape, 0).astype(jnp.float32)
    sel = jnp.where(x == m, iota, float(NUM_CLS))
    pred = jnp.min(sel, axis=0).astype(jnp.int32)
    out_ref[...] = t * NUM_CLS + pred


def _hist_body(nc, perw, keys_hbm, out_hbm, kv, hist):
    wid = lax.axis_index("s") * nc + lax.axis_index("c")
    base = wid * perw

    zeros16 = jnp.zeros((16,), jnp.float32)

    def zero_body(i, carry):
        hist[pl.ds(i * 16, 16)] = zeros16
        return carry

    lax.fori_loop(0, HIST // 16, zero_body, 0)

    pltpu.sync_copy(keys_hbm.at[pl.ds(base, perw)], kv)

    ones16 = jnp.ones((16,), jnp.float32)

    def body(i, carry):
        k = kv[pl.ds(i * 16, 16)]
        plsc.addupdate_scatter(hist, [k], ones16)
        return carry

    lax.fori_loop(0, perw // 16, body, 0)

    pltpu.sync_copy(hist, out_hbm.at[wid])


def _f1_body(h_ref, o_ref):
    h = h_ref[...]
    cm = jnp.sum(h, axis=0)
    ii = lax.broadcasted_iota(jnp.int32, (NUM_CLS, NUM_CLS), 0)
    jj = lax.broadcasted_iota(jnp.int32, (NUM_CLS, NUM_CLS), 1)
    diag = jnp.sum(jnp.where(ii == jj, cm, 0.0), axis=1, keepdims=True)
    rows = jnp.sum(cm, axis=1, keepdims=True)
    ones_col = jnp.ones((NUM_CLS, 1), jnp.float32)
    cols = lax.dot_general(cm, ones_col, (((0,), (0,)), ((), ())),
                           preferred_element_type=jnp.float32)
    precision = diag / (rows + 1e-12)
    recall = diag / (cols + 1e-12)
    f1 = 2.0 * precision * recall / (precision + recall + 1e-12)
    o_ref[...] = jnp.sum(f1, axis=(0, 1), keepdims=True) / NUM_CLS


def kernel(y_pred, y_true):
    n, c = y_pred.shape
    assert c == NUM_CLS

    blk = 2048
    grid = n // blk
    keys = pl.pallas_call(
        _argmax_body,
        grid=(grid,),
        in_specs=[
            pl.BlockSpec((c, blk), lambda i: (0, i)),
            pl.BlockSpec((blk,), lambda i: (i,)),
        ],
        out_specs=pl.BlockSpec((blk,), lambda i: (i,)),
        out_shape=jax.ShapeDtypeStruct((n,), jnp.int32),
    )(y_pred.T, y_true)

    mesh = plsc.VectorSubcoreMesh(core_axis_name="c", subcore_axis_name="s")
    nw = mesh.num_cores * mesh.num_subcores
    perw = n // nw
    hist_kernel = pl.kernel(
        functools.partial(_hist_body, mesh.num_cores, perw),
        out_type=jax.ShapeDtypeStruct((nw, HIST), jnp.float32),
        mesh=mesh,
        scratch_types=[
            pltpu.VMEM((perw,), jnp.int32),
            pltpu.VMEM((HIST,), jnp.float32),
        ],
        compiler_params=pltpu.CompilerParams(needs_layout_passes=False),
    )
    hists = hist_kernel(keys)

    out = pl.pallas_call(
        _f1_body,
        out_shape=jax.ShapeDtypeStruct((1, 1), jnp.float32),
    )(hists.reshape(nw, NUM_CLS, NUM_CLS))
    return out[0, 0]


# speedup vs baseline: 5.8074x; 1.8442x over previous
import functools

import jax
import jax.numpy as jnp
from jax import lax
from jax.experimental import pallas as pl
from jax.experimental.pallas import tpu as pltpu
from jax.experimental.pallas import tpu_sc as plsc

NUM_CLS = 100
HIST = NUM_CLS * NUM_CLS


def _argmax_body(ypt_ref, yt_ref, out_ref):
    x = ypt_ref[...]
    t = yt_ref[...]
    m = jnp.max(x, axis=0, keepdims=True)
    iota = lax.broadcasted_iota(jnp.int32, x.shape, 0).astype(jnp.float32)
    sel = jnp.where(x == m, iota, float(NUM_CLS))
    pred = jnp.min(sel, axis=0).astype(jnp.int32)
    out_ref[...] = t * NUM_CLS + pred


def _hist_body(nc, perw, keys_hbm, out_hbm, kv, hist):
    wid = lax.axis_index("s") * nc + lax.axis_index("c")
    base = wid * perw

    zeros16 = jnp.zeros((16,), jnp.float32)

    def zero_body(i, carry):
        hist[pl.ds(i * 16, 16)] = zeros16
        return carry

    lax.fori_loop(0, HIST // 16, zero_body, 0)

    pltpu.sync_copy(keys_hbm.at[pl.ds(base, perw)], kv)

    ones16 = jnp.ones((16,), jnp.float32)

    def body(i, carry):
        k = kv[pl.ds(i * 16, 16)]
        plsc.addupdate_scatter(hist, [k], ones16)
        return carry

    lax.fori_loop(0, perw // 16, body, 0)

    pltpu.sync_copy(hist, out_hbm.at[wid])


def _f1_body(h_ref, o_ref):
    h = h_ref[...]
    cm = jnp.sum(h, axis=0)
    ii = lax.broadcasted_iota(jnp.int32, (NUM_CLS, NUM_CLS), 0)
    jj = lax.broadcasted_iota(jnp.int32, (NUM_CLS, NUM_CLS), 1)
    diag = jnp.sum(jnp.where(ii == jj, cm, 0.0), axis=1, keepdims=True)
    rows = jnp.sum(cm, axis=1, keepdims=True)
    ones_col = jnp.ones((NUM_CLS, 1), jnp.float32)
    cols = lax.dot_general(cm, ones_col, (((0,), (0,)), ((), ())),
                           preferred_element_type=jnp.float32)
    precision = diag / (rows + 1e-12)
    recall = diag / (cols + 1e-12)
    f1 = 2.0 * precision * recall / (precision + recall + 1e-12)
    o_ref[...] = jnp.sum(f1, axis=(0, 1), keepdims=True) / NUM_CLS


def kernel(y_pred, y_true):
    n, c = y_pred.shape
    assert c == NUM_CLS

    blk = 8192
    grid = n // blk
    keys = pl.pallas_call(
        _argmax_body,
        grid=(grid,),
        in_specs=[
            pl.BlockSpec((c, blk), lambda i: (0, i)),
            pl.BlockSpec((blk,), lambda i: (i,)),
        ],
        out_specs=pl.BlockSpec((blk,), lambda i: (i,)),
        out_shape=jax.ShapeDtypeStruct((n,), jnp.int32),
    )(y_pred.T, y_true)

    mesh = plsc.VectorSubcoreMesh(core_axis_name="c", subcore_axis_name="s")
    nw = mesh.num_cores * mesh.num_subcores
    perw = n // nw
    hist_kernel = pl.kernel(
        functools.partial(_hist_body, mesh.num_cores, perw),
        out_type=jax.ShapeDtypeStruct((nw, HIST), jnp.float32),
        mesh=mesh,
        scratch_types=[
            pltpu.VMEM((perw,), jnp.int32),
            pltpu.VMEM((HIST,), jnp.float32),
        ],
        compiler_params=pltpu.CompilerParams(needs_layout_passes=False),
    )
    hists = hist_kernel(keys)

    out = pl.pallas_call(
        _f1_body,
        out_shape=jax.ShapeDtypeStruct((1, 1), jnp.float32),
    )(hists.reshape(nw, NUM_CLS, NUM_CLS))
    return out[0, 0]


# speedup vs baseline: 6.8681x; 1.1826x over previous
import functools

import jax
import jax.numpy as jnp
from jax import lax
from jax.experimental import pallas as pl
from jax.experimental.pallas import tpu as pltpu
from jax.experimental.pallas import tpu_sc as plsc

NUM_CLS = 100
HIST = NUM_CLS * NUM_CLS


def _argmax_body(ypt_ref, yt_ref, out_ref):
    x = ypt_ref[...]
    t = yt_ref[...]
    m = jnp.max(x, axis=0, keepdims=True)
    iota = lax.broadcasted_iota(jnp.int32, x.shape, 0).astype(jnp.float32)
    sel = jnp.where(x == m, iota, float(NUM_CLS))
    pred = jnp.min(sel, axis=0).astype(jnp.int32)
    out_ref[...] = t * NUM_CLS + pred


def _hist_body(nc, perw, keys_hbm, out_hbm, kv, hist):
    wid = lax.axis_index("s") * nc + lax.axis_index("c")
    base = wid * perw

    zeros16 = jnp.zeros((16,), jnp.float32)

    def zero_body(i, carry):
        hist[pl.ds(i * 16, 16)] = zeros16
        return carry

    lax.fori_loop(0, HIST // 16, zero_body, 0)

    pltpu.sync_copy(keys_hbm.at[pl.ds(base, perw)], kv)

    ones16 = jnp.ones((16,), jnp.float32)

    def body(i, carry):
        k = kv[pl.ds(i * 16, 16)]
        plsc.addupdate_scatter(hist, [k], ones16)
        return carry

    lax.fori_loop(0, perw // 16, body, 0)

    pltpu.sync_copy(hist, out_hbm.at[wid])


def _f1_body(h_ref, o_ref):
    h = h_ref[...]
    cm = jnp.sum(h, axis=0)
    ii = lax.broadcasted_iota(jnp.int32, (NUM_CLS, NUM_CLS), 0)
    jj = lax.broadcasted_iota(jnp.int32, (NUM_CLS, NUM_CLS), 1)
    diag = jnp.sum(jnp.where(ii == jj, cm, 0.0), axis=1, keepdims=True)
    rows = jnp.sum(cm, axis=1, keepdims=True)
    ones_col = jnp.ones((NUM_CLS, 1), jnp.float32)
    cols = lax.dot_general(cm, ones_col, (((0,), (0,)), ((), ())),
                           preferred_element_type=jnp.float32)
    precision = diag / (rows + 1e-12)
    recall = diag / (cols + 1e-12)
    f1 = 2.0 * precision * recall / (precision + recall + 1e-12)
    o_ref[...] = jnp.sum(f1, axis=(0, 1), keepdims=True) / NUM_CLS


def kernel(y_pred, y_true):
    n, c = y_pred.shape
    assert c == NUM_CLS

    blk = 16384
    grid = n // blk
    keys = pl.pallas_call(
        _argmax_body,
        grid=(grid,),
        in_specs=[
            pl.BlockSpec((c, blk), lambda i: (0, i)),
            pl.BlockSpec((blk,), lambda i: (i,)),
        ],
        out_specs=pl.BlockSpec((blk,), lambda i: (i,)),
        out_shape=jax.ShapeDtypeStruct((n,), jnp.int32),
    )(y_pred.T, y_true)

    mesh = plsc.VectorSubcoreMesh(core_axis_name="c", subcore_axis_name="s")
    nw = mesh.num_cores * mesh.num_subcores
    perw = n // nw
    hist_kernel = pl.kernel(
        functools.partial(_hist_body, mesh.num_cores, perw),
        out_type=jax.ShapeDtypeStruct((nw, HIST), jnp.float32),
        mesh=mesh,
        scratch_types=[
            pltpu.VMEM((perw,), jnp.int32),
            pltpu.VMEM((HIST,), jnp.float32),
        ],
        compiler_params=pltpu.CompilerParams(needs_layout_passes=False),
    )
    hists = hist_kernel(keys)

    out = pl.pallas_call(
        _f1_body,
        out_shape=jax.ShapeDtypeStruct((1, 1), jnp.float32),
    )(hists.reshape(nw, NUM_CLS, NUM_CLS))
    return out[0, 0]


# speedup vs baseline: 7.2466x; 1.0551x over previous
import functools

import jax
import jax.numpy as jnp
from jax import lax
from jax.experimental import pallas as pl
from jax.experimental.pallas import tpu as pltpu
from jax.experimental.pallas import tpu_sc as plsc

NUM_CLS = 100
HIST = NUM_CLS * NUM_CLS


def _argmax_body(ypt_ref, yt_ref, out_ref):
    x = ypt_ref[...]
    t = yt_ref[...]
    m = jnp.max(x, axis=0, keepdims=True)
    iota = lax.broadcasted_iota(jnp.int32, x.shape, 0).astype(jnp.float32)
    sel = jnp.where(x == m, iota, float(NUM_CLS))
    pred = jnp.min(sel, axis=0).astype(jnp.int32)
    out_ref[...] = t * NUM_CLS + pred


def _hist_body(nc, perw, keys_hbm, out_hbm, kv, hist):
    wid = lax.axis_index("s") * nc + lax.axis_index("c")
    base = wid * perw

    zeros16 = jnp.zeros((16,), jnp.float32)

    def zero_body(i, carry):
        hist[pl.ds(i * 16, 16)] = zeros16
        return carry

    lax.fori_loop(0, HIST // 16, zero_body, 0)

    pltpu.sync_copy(keys_hbm.at[pl.ds(base, perw)], kv)

    ones16 = jnp.ones((16,), jnp.float32)

    def body(i, carry):
        k = kv[pl.ds(i * 16, 16)]
        plsc.addupdate_scatter(hist, [k], ones16)
        return carry

    lax.fori_loop(0, perw // 16, body, 0)

    pltpu.sync_copy(hist, out_hbm.at[wid])


def _f1_body(h_ref, o_ref):
    h = h_ref[...]
    cm = jnp.sum(h, axis=0)
    ii = lax.broadcasted_iota(jnp.int32, (NUM_CLS, NUM_CLS), 0)
    jj = lax.broadcasted_iota(jnp.int32, (NUM_CLS, NUM_CLS), 1)
    diag = jnp.sum(jnp.where(ii == jj, cm, 0.0), axis=1, keepdims=True)
    rows = jnp.sum(cm, axis=1, keepdims=True)
    ones_col = jnp.ones((NUM_CLS, 1), jnp.float32)
    cols = lax.dot_general(cm, ones_col, (((0,), (0,)), ((), ())),
                           preferred_element_type=jnp.float32)
    precision = diag / (rows + 1e-12)
    recall = diag / (cols + 1e-12)
    f1 = 2.0 * precision * recall / (precision + recall + 1e-12)
    o_ref[...] = jnp.sum(f1, axis=(0, 1), keepdims=True) / NUM_CLS


def kernel(y_pred, y_true):
    n, c = y_pred.shape
    assert c == NUM_CLS

    blk = 32768
    grid = n // blk
    keys = pl.pallas_call(
        _argmax_body,
        grid=(grid,),
        in_specs=[
            pl.BlockSpec((c, blk), lambda i: (0, i)),
            pl.BlockSpec((blk,), lambda i: (i,)),
        ],
        out_specs=pl.BlockSpec((blk,), lambda i: (i,)),
        out_shape=jax.ShapeDtypeStruct((n,), jnp.int32),
    )(y_pred.T, y_true)

    mesh = plsc.VectorSubcoreMesh(core_axis_name="c", subcore_axis_name="s")
    nw = mesh.num_cores * mesh.num_subcores
    perw = n // nw
    hist_kernel = pl.kernel(
        functools.partial(_hist_body, mesh.num_cores, perw),
        out_type=jax.ShapeDtypeStruct((nw, HIST), jnp.float32),
        mesh=mesh,
        scratch_types=[
            pltpu.VMEM((perw,), jnp.int32),
            pltpu.VMEM((HIST,), jnp.float32),
        ],
        compiler_params=pltpu.CompilerParams(needs_layout_passes=False),
    )
    hists = hist_kernel(keys)

    out = pl.pallas_call(
        _f1_body,
        out_shape=jax.ShapeDtypeStruct((1, 1), jnp.float32),
    )(hists.reshape(nw, NUM_CLS, NUM_CLS))
    return out[0, 0]


# speedup vs baseline: 7.5023x; 1.0353x over previous
import functools

import jax
import jax.numpy as jnp
from jax import lax
from jax.experimental import pallas as pl
from jax.experimental.pallas import tpu as pltpu
from jax.experimental.pallas import tpu_sc as plsc

NUM_CLS = 100
HIST = NUM_CLS * NUM_CLS


def _argmax_body(ypt_ref, yt_ref, out_ref):
    x = ypt_ref[...]
    t = yt_ref[...]
    m = jnp.max(x, axis=0, keepdims=True)
    iota = lax.broadcasted_iota(jnp.int32, x.shape, 0).astype(jnp.float32)
    sel = jnp.where(x == m, iota, float(NUM_CLS))
    pred = jnp.min(sel, axis=0).astype(jnp.int32)
    out_ref[...] = t * 128 + pred


def _hist_body(nc, perw, keys_hbm, out_hbm, kv, hist):
    wid = lax.axis_index("s") * nc + lax.axis_index("c")
    base = wid * perw

    zeros16 = jnp.zeros((16,), jnp.float32)

    def zero_body(i, carry):
        for j in range(128 // 16):
            hist[i, pl.ds(j * 16, 16)] = zeros16
        return carry

    lax.fori_loop(0, NUM_CLS, zero_body, 0)

    pltpu.sync_copy(keys_hbm.at[pl.ds(base, perw)], kv)

    ones16 = jnp.ones((16,), jnp.float32)
    unroll = 4

    def body(i, carry):
        for j in range(unroll):
            k = kv[pl.ds((i * unroll + j) * 16, 16)]
            hi = k >> 7
            lo = k & 127
            plsc.addupdate_scatter(hist, [hi, lo], ones16)
        return carry

    lax.fori_loop(0, perw // (16 * unroll), body, 0)

    pltpu.sync_copy(hist, out_hbm.at[wid])


def _f1_body(h_ref, o_ref):
    h = h_ref[...]
    cm = jnp.sum(h, axis=0)
    ii = lax.broadcasted_iota(jnp.int32, cm.shape, 0)
    jj = lax.broadcasted_iota(jnp.int32, cm.shape, 1)
    diag = jnp.sum(jnp.where(ii == jj, cm, 0.0), axis=1, keepdims=True)
    rows = jnp.sum(cm, axis=1, keepdims=True)
    ones_col = jnp.ones((NUM_CLS, 1), jnp.float32)
    cols = lax.dot_general(cm, ones_col, (((0,), (0,)), ((), ())),
                           preferred_element_type=jnp.float32)
    cols = lax.slice(cols, (0, 0), (NUM_CLS, 1))
    precision = diag / (rows + 1e-12)
    recall = diag / (cols + 1e-12)
    f1 = 2.0 * precision * recall / (precision + recall + 1e-12)
    o_ref[...] = jnp.sum(f1, axis=(0, 1), keepdims=True) / NUM_CLS


def kernel(y_pred, y_true):
    n, c = y_pred.shape
    assert c == NUM_CLS

    blk = 32768
    grid = n // blk
    keys = pl.pallas_call(
        _argmax_body,
        grid=(grid,),
        in_specs=[
            pl.BlockSpec((c, blk), lambda i: (0, i)),
            pl.BlockSpec((blk,), lambda i: (i,)),
        ],
        out_specs=pl.BlockSpec((blk,), lambda i: (i,)),
        out_shape=jax.ShapeDtypeStruct((n,), jnp.int32),
    )(y_pred.T, y_true)

    mesh = plsc.VectorSubcoreMesh(core_axis_name="c", subcore_axis_name="s")
    nw = mesh.num_cores * mesh.num_subcores
    perw = n // nw
    hist_kernel = pl.kernel(
        functools.partial(_hist_body, mesh.num_cores, perw),
        out_type=jax.ShapeDtypeStruct((nw, NUM_CLS, 128), jnp.float32),
        mesh=mesh,
        scratch_types=[
            pltpu.VMEM((perw,), jnp.int32),
            pltpu.VMEM((NUM_CLS, 128), jnp.float32),
        ],
        compiler_params=pltpu.CompilerParams(needs_layout_passes=False),
    )
    hists = hist_kernel(keys)

    out = pl.pallas_call(
        _f1_body,
        out_shape=jax.ShapeDtypeStruct((1, 1), jnp.float32),
    )(hists)
    return out[0, 0]
